# Initial kernel scaffold; baseline (speedup 1.0000x reference)
import jax
import jax.numpy as jnp
from jax.experimental import pallas as pl


def kernel(x, edge_index, edge_att, W1, b1, W2, b2):
    raise NotImplementedError("write your pallas kernel here")


# speedup vs baseline: 3.9028x; 3.9028x over previous
import jax
import jax.numpy as jnp
from jax import lax
from jax.experimental import pallas as pl
from jax.experimental.pallas import tpu as pltpu
from jax.experimental.pallas import tpu_sc as plsc

NC = 2
NS = 16
L = 16
CHUNK = 128
CB = 32


def _sc_pass(x_rows, src_p, dst_p, att_p):
  n = x_rows.shape[0]
  C = src_p.shape[1]
  half = n // NC
  rpt = ((-(-half // NS)) + 7) // 8 * 8

  out_type = (jax.ShapeDtypeStruct((n, 128), jnp.float32),
              jax.ShapeDtypeStruct((n, 16), jnp.float32))

  scratch = dict(
      src_v=pltpu.VMEM((CB, CHUNK), jnp.int32),
      dst_v=pltpu.VMEM((CB, CHUNK), jnp.int32),
      att_v=pltpu.VMEM((CB, CHUNK), jnp.float32),
      dstloc_v=pltpu.VMEM((1, CHUNK), jnp.int32),
      rows_v=pltpu.VMEM((CHUNK, 128), jnp.float32),
      attrow_v=pltpu.VMEM((CHUNK, 16), jnp.float32),
      s_sh=pltpu.VMEM_SHARED((half, 128), jnp.float32),
      den_sh=pltpu.VMEM_SHARED((half, 16), jnp.float32),
      sem=pltpu.SemaphoreType.DMA,
  )

  mesh = plsc.VectorSubcoreMesh(core_axis_name="c", subcore_axis_name="s",
                                num_cores=NC, num_subcores=NS)

  def body(x_hbm, src_hbm, dst_hbm, att_hbm, s_out, den_out, *, src_v,
           dst_v, att_v, dstloc_v, rows_v, attrow_v, s_sh, den_sh, sem):
    cid = lax.axis_index("c")
    sid = lax.axis_index("s")
    base = cid * half

    zv = jnp.zeros((L,), jnp.float32)

    def zero_rows(i, _):
      for v in range(128 // L):
        rows_v[i, pl.ds(v * L, L)] = zv
      attrow_v[i, :] = zv
      return 0

    lax.fori_loop(0, CHUNK, zero_rows, 0)

    off = pl.multiple_of(jnp.minimum(sid * rpt, half - rpt), 8)
    nblk = rpt // CHUNK
    rem = rpt - nblk * CHUNK
    for b in range(nblk):
      pltpu.sync_copy(rows_v, s_sh.at[pl.ds(off + b * CHUNK, CHUNK)])
      pltpu.sync_copy(attrow_v, den_sh.at[pl.ds(off + b * CHUNK, CHUNK)])
    if rem:
      pltpu.sync_copy(rows_v.at[pl.ds(0, rem)],
                      s_sh.at[pl.ds(off + nblk * CHUNK, rem)])
      pltpu.sync_copy(attrow_v.at[pl.ds(0, rem)],
                      den_sh.at[pl.ds(off + nblk * CHUNK, rem)])

    plsc.subcore_barrier()

    def block_body(nb, _):
      boff = pl.multiple_of(nb * CB, 8)
      pltpu.sync_copy(src_hbm.at[sid, pl.ds(boff, CB)], src_v)
      pltpu.sync_copy(dst_hbm.at[sid, pl.ds(boff, CB)], dst_v)
      pltpu.sync_copy(att_hbm.at[sid, pl.ds(boff, CB)], att_v)
      lax.fori_loop(0, CB, chunk_body, 0)
      return 0

    def chunk_body(c, _):
      pltpu.async_copy(x_hbm.at[src_v.at[c]], rows_v, sem).wait()

      def group(j8, _):
        sl = pl.ds(j8 * L, L)
        a = att_v[c, sl]
        s = src_v[c, sl]
        d = dst_v[c, sl]
        dl = d - base
        inr = (dl >= 0) & (dl < half)
        wv = jnp.where(inr & (s != d), jnp.exp(a), 0.0)
        am = jnp.where(inr, a, 0.0)
        dstloc_v[0, sl] = jnp.where(inr, dl, 0)
        for l in range(L):
          row = j8 * L + l
          attrow_v[row, :] = jnp.full((L,), am[l], jnp.float32)
          wb = jnp.full((L,), wv[l], jnp.float32)
          for v in range(128 // L):
            slv = pl.ds(v * L, L)
            rows_v[row, slv] = rows_v[row, slv] * wb
        return 0

      lax.fori_loop(0, CHUNK // L, group, 0)

      pltpu.sync_copy(rows_v, s_sh.at[dstloc_v.at[0]], add=True)
      pltpu.sync_copy(attrow_v, den_sh.at[dstloc_v.at[0]], add=True)
      return 0

    lax.fori_loop(0, C // CB, block_body, 0)

    plsc.subcore_barrier()

    goff = pl.multiple_of(base + off, 8)
    pltpu.sync_copy(s_sh.at[pl.ds(off, rpt)], s_out.at[pl.ds(goff, rpt)])
    pltpu.sync_copy(den_sh.at[pl.ds(off, rpt)], den_out.at[pl.ds(goff, rpt)])

  fn = pl.kernel(body, out_type=out_type, mesh=mesh, scratch_types=scratch)
  return fn(x_rows, src_p, dst_p, att_p)


def _tc_mid(s1, x, denom, W1, b1, W2):
  n = x.shape[0]
  B = 1000
  grid = (n // B,)

  def body(s1_ref, x_ref, den_ref, w1_ref, b1_ref, w2_ref, out_ref):
    r = jnp.exp(-den_ref[...])
    z1 = (s1_ref[...] + x_ref[...]) * r
    g = jnp.maximum(
        jnp.dot(z1, w1_ref[...], preferred_element_type=jnp.float32)
        + b1_ref[...], 0.0)
    out_ref[...] = jnp.dot(g, w2_ref[...], preferred_element_type=jnp.float32)

  return pl.pallas_call(
      body,
      grid=grid,
      in_specs=[
          pl.BlockSpec((B, 128), lambda i: (i, 0)),
          pl.BlockSpec((B, 128), lambda i: (i, 0)),
          pl.BlockSpec((B, 1), lambda i: (i, 0)),
          pl.BlockSpec((128, 256), lambda i: (0, 0)),
          pl.BlockSpec((1, 256), lambda i: (0, 0)),
          pl.BlockSpec((256, 128), lambda i: (0, 0)),
      ],
      out_specs=pl.BlockSpec((B, 128), lambda i: (i, 0)),
      out_shape=jax.ShapeDtypeStruct((n, 128), jnp.float32),
  )(s1, x, denom, W1, b1, W2)


def _tc_final(s2, h2, denom, b2):
  n = h2.shape[0]
  B = 1000
  grid = (n // B,)

  def body(s2_ref, h2_ref, den_ref, b2_ref, out_ref):
    r = jnp.exp(-den_ref[...])
    out_ref[...] = (s2_ref[...] + h2_ref[...]) * r + b2_ref[...]

  return pl.pallas_call(
      body,
      grid=grid,
      in_specs=[
          pl.BlockSpec((B, 128), lambda i: (i, 0)),
          pl.BlockSpec((B, 128), lambda i: (i, 0)),
          pl.BlockSpec((B, 1), lambda i: (i, 0)),
          pl.BlockSpec((1, 128), lambda i: (0, 0)),
      ],
      out_specs=pl.BlockSpec((B, 128), lambda i: (i, 0)),
      out_shape=jax.ShapeDtypeStruct((n, 128), jnp.float32),
  )(s2, h2, denom, b2)


def kernel(x, edge_index, edge_att, W1, b1, W2, b2):
  n = x.shape[0]
  e = edge_index.shape[1]
  src = edge_index[0]
  dst = edge_index[1]

  ew = -(-e // NS)
  C = -(-(-(-ew // CHUNK)) // CB) * CB
  ep = C * CHUNK
  pad_tail = NS * ew - e

  def part(a, fill):
    a = jnp.pad(a, (0, pad_tail), constant_values=fill).reshape(NS, ew)
    a = jnp.pad(a, ((0, 0), (0, ep - ew)), constant_values=fill)
    return a.reshape(NS, C, CHUNK)

  src_p = part(src, 0)
  dst_p = part(dst, 0)
  att_p = part(edge_att, 0.0)

  s1, den = _sc_pass(x, src_p, dst_p, att_p)
  denom = den[:, 0].reshape(n, 1)

  h2 = _tc_mid(s1, x, denom, W1, b1.reshape(1, -1), W2)

  s2, _ = _sc_pass(h2, src_p, dst_p, att_p)

  return _tc_final(s2, h2, denom, b2.reshape(1, -1))
